# baseline (speedup 1.0000x reference)
import functools

import jax
import jax.numpy as jnp
from jax import lax
from jax.experimental import pallas as pl
from jax.experimental.pallas import tpu as pltpu
from jax.experimental.pallas import tpu_sc as plsc

NC, NS, LANES = 2, 16, 16
NW = NC * NS
B = 1024
T = 1000
TPAD = 1024
CHUNK = 128
NCHUNK = TPAD // CHUNK
D = 32
BPW = B // NW
NGRP = TPAD // LANES
NGRP_FULL = T // LANES
BIG = 2**30


def _fire(table_hbm, idx_all, rows, sem, i):
    return [
        pltpu.async_copy(table_hbm.at[idx_all.at[i, j]],
                         rows.at[pl.ds(j * CHUNK, CHUNK)], sem)
        for j in range(NCHUNK)
    ]


def _drain(table_hbm, idx_all, rows, sem, i):
    for j in range(NCHUNK):
        pltpu.make_async_copy(table_hbm.at[idx_all.at[i, j]],
                              rows.at[pl.ds(j * CHUNK, CHUNK)], sem).wait()


def _batch_compute(rows, normbuf, feat_v, i, iota):
    zeros = jnp.zeros((LANES,), jnp.float32)
    unroll = 8

    def body1(it, carry):
        s = list(carry)
        for u in range(unroll):
            r = it * unroll + u
            a = rows[r, pl.ds(0, LANES)]
            b = rows[r, pl.ds(LANES, LANES)]
            s[2 * (u % 2)] = s[2 * (u % 2)] + a
            s[2 * (u % 2) + 1] = s[2 * (u % 2) + 1] + b
            c = a * a + b * b
            normbuf[r, pl.ds(0, LANES)] = jnp.cumsum(c)
        return tuple(s)

    s = lax.fori_loop(0, T // unroll, body1, (zeros,) * 4)
    t0 = s[0] + s[2]
    t1 = s[1] + s[3]

    lane15 = jnp.full((LANES,), LANES - 1, jnp.int32)

    def body2(g, carry):
        m16, bi16 = carry
        row_idx = g * LANES + iota
        nv = plsc.load_gather(normbuf, [row_idx, lane15])
        pred = nv > m16
        m16 = jnp.where(pred, nv, m16)
        bi16 = jnp.where(pred, row_idx, bi16)
        return m16, bi16

    m16, bi16 = lax.fori_loop(0, NGRP_FULL, body2,
                              (jnp.full((LANES,), -1.0, jnp.float32),
                               jnp.zeros((LANES,), jnp.int32)))
    row_idx = jnp.int32(NGRP_FULL * LANES) + iota
    nv = plsc.load_gather(normbuf, [row_idx, lane15])
    nv = jnp.where(iota < 8, nv, jnp.float32(-1.0))
    pred = nv > m16
    m16 = jnp.where(pred, nv, m16)
    bi16 = jnp.where(pred, row_idx, bi16)

    mmax = jnp.max(m16)
    cand = jnp.where(m16 == mmax, bi16, jnp.int32(BIG))
    bi = jnp.min(cand)
    best0 = rows[bi, pl.ds(0, LANES)]
    best1 = rows[bi, pl.ds(LANES, LANES)]

    inv = jnp.float32(1.0 / T)
    off = i * (2 * D)
    feat_v[pl.ds(off, LANES)] = t0 * inv
    feat_v[pl.ds(off + LANES, LANES)] = t1 * inv
    feat_v[pl.ds(off + 2 * LANES, LANES)] = best0
    feat_v[pl.ds(off + 3 * LANES, LANES)] = best1


def _sc_body(idx_hbm, table_hbm, w_hbm, b_hbm, out_hbm,
             idx_all, rows_a, normbuf, feat_v, w_v, bias_v,
             out_v, sem_a):
    wid = lax.axis_index("s") * NC + lax.axis_index("c")
    base = wid * BPW
    iota = lax.iota(jnp.int32, LANES)

    pltpu.sync_copy(w_hbm, w_v)
    pltpu.sync_copy(b_hbm, bias_v)

    def batch_body(i, carry):
        b = base + i
        pltpu.sync_copy(idx_hbm.at[b], idx_all)
        copies = [
            pltpu.async_copy(table_hbm.at[idx_all.at[j]],
                             rows_a.at[pl.ds(j * CHUNK, CHUNK)], sem_a)
            for j in range(NCHUNK)
        ]
        for c in copies:
            c.wait()
        _batch_compute(rows_a, normbuf, feat_v, i, iota)
        return carry

    lax.fori_loop(0, BPW, batch_body, 0)

    bvec = bias_v[pl.ds(0, LANES)]
    b0 = bvec[0]
    b1 = bvec[1]
    w0vecs = [w_v[pl.ds(k * LANES, LANES)] for k in range(2 * D // LANES)]
    w1vecs = [w_v[pl.ds(2 * D + k * LANES, LANES)] for k in range(2 * D // LANES)]
    iota_feat = iota * (2 * D)
    for g in range(BPW // LANES):
        acc0 = jnp.broadcast_to(b0, (LANES,))
        acc1 = jnp.broadcast_to(b1, (LANES,))
        gbase = g * LANES * (2 * D)
        for d in range(2 * D):
            v = plsc.load_gather(feat_v, [iota_feat + (gbase + d)])
            acc0 = acc0 + v * w0vecs[d // LANES][d % LANES]
            acc1 = acc1 + v * w1vecs[d // LANES][d % LANES]
        p0 = 1.0 / (1.0 + jnp.exp(-acc0))
        p1 = 1.0 / (1.0 + jnp.exp(-acc1))
        row_idx = g * LANES + iota
        plsc.store_scatter(out_v, [row_idx, jnp.zeros((LANES,), jnp.int32)], p0)
        plsc.store_scatter(out_v, [row_idx, jnp.ones((LANES,), jnp.int32)], p1)

    pltpu.sync_copy(out_v, out_hbm.at[pl.ds(base, BPW)])


@jax.jit
def _logreg_sc(idx3, table, wflat, bpad):
    mesh = plsc.VectorSubcoreMesh(core_axis_name="c", subcore_axis_name="s",
                                  num_cores=NC, num_subcores=NS)
    fn = pl.kernel(
        _sc_body,
        out_type=jax.ShapeDtypeStruct((B, 2), jnp.float32),
        mesh=mesh,
        compiler_params=pltpu.CompilerParams(needs_layout_passes=False,
                                             use_tc_tiling_on_sc=False),
        scratch_types=[
            pltpu.VMEM((NCHUNK, CHUNK), jnp.int32),
            pltpu.VMEM((TPAD, D), jnp.float32),
            pltpu.VMEM((TPAD, LANES), jnp.float32),
            pltpu.VMEM((BPW * 2 * D,), jnp.float32),
            pltpu.VMEM((2 * 2 * D,), jnp.float32),
            pltpu.VMEM((LANES,), jnp.float32),
            pltpu.VMEM((BPW, 2), jnp.float32),
            pltpu.SemaphoreType.DMA,
        ],
    )
    return fn(idx3, table, wflat, bpad)


def kernel(indices, embedding_matrix, dense_W, dense_b):
    idx = indices.reshape(B, T).astype(jnp.int32)
    idxp = jnp.pad(idx, ((0, 0), (0, TPAD - T)))
    idx3 = idxp.reshape(B, NCHUNK, CHUNK)
    wflat = dense_W.astype(jnp.float32).T.reshape(2 * 2 * D)
    bpad = jnp.pad(dense_b.astype(jnp.float32), (0, LANES - 2))
    return _logreg_sc(idx3, embedding_matrix, wflat, bpad)


# speedup vs baseline: 1.2611x; 1.2611x over previous
import functools

import jax
import jax.numpy as jnp
from jax import lax
from jax.experimental import pallas as pl
from jax.experimental.pallas import tpu as pltpu
from jax.experimental.pallas import tpu_sc as plsc

NC, NS, LANES = 2, 16, 16
NW = NC * NS
B = 1024
T = 1000
TPAD = 1024
CHUNK = 128
NCHUNK = TPAD // CHUNK
D = 32
BPW = B // NW
ROW_UNROLL = 8


def _sc_body(idx_hbm, table_hbm, w_hbm, b_hbm, out_hbm,
             idx_v, rows_v, feat_v, w_v, bias_v, out_v, sem):
    wid = lax.axis_index("s") * NC + lax.axis_index("c")
    base = wid * BPW
    iota = lax.iota(jnp.int32, LANES)

    pltpu.sync_copy(w_hbm, w_v)
    pltpu.sync_copy(b_hbm, bias_v)

    def batch_body(i, carry):
        b = base + i
        pltpu.sync_copy(idx_hbm.at[b], idx_v)
        copies = [
            pltpu.async_copy(table_hbm.at[idx_v.at[j]],
                             rows_v.at[pl.ds(j * CHUNK, CHUNK)], sem)
            for j in range(NCHUNK)
        ]
        for c in copies:
            c.wait()

        zeros = jnp.zeros((LANES,), jnp.float32)
        init = (zeros, zeros, jnp.float32(-1.0), zeros, zeros)

        def row_body(it, c):
            s0, s1, m, bv0, bv1 = c
            for u in range(ROW_UNROLL):
                r = it * ROW_UNROLL + u
                a = rows_v[r, pl.ds(0, LANES)]
                bb = rows_v[r, pl.ds(LANES, LANES)]
                s0 = s0 + a
                s1 = s1 + bb
                nsq = jnp.sum(a * a + bb * bb)
                pred = nsq > m
                m = jnp.where(pred, nsq, m)
                pv = jnp.broadcast_to(pred, (LANES,))
                bv0 = jnp.where(pv, a, bv0)
                bv1 = jnp.where(pv, bb, bv1)
            return (s0, s1, m, bv0, bv1)

        s0, s1, m, bv0, bv1 = lax.fori_loop(0, T // ROW_UNROLL, row_body, init)
        inv = jnp.float32(1.0 / T)
        off = i * (2 * D)
        feat_v[pl.ds(off, LANES)] = s0 * inv
        feat_v[pl.ds(off + LANES, LANES)] = s1 * inv
        feat_v[pl.ds(off + 2 * LANES, LANES)] = bv0
        feat_v[pl.ds(off + 3 * LANES, LANES)] = bv1
        return carry

    lax.fori_loop(0, BPW, batch_body, 0)

    bvec = bias_v[pl.ds(0, LANES)]
    b0 = bvec[0]
    b1 = bvec[1]
    w0vecs = [w_v[pl.ds(k * LANES, LANES)] for k in range(2 * D // LANES)]
    w1vecs = [w_v[pl.ds(2 * D + k * LANES, LANES)] for k in range(2 * D // LANES)]
    iota_feat = iota * (2 * D)
    for g in range(BPW // LANES):
        acc0 = jnp.broadcast_to(b0, (LANES,))
        acc1 = jnp.broadcast_to(b1, (LANES,))
        gbase = g * LANES * (2 * D)
        for d in range(2 * D):
            v = plsc.load_gather(feat_v, [iota_feat + (gbase + d)])
            acc0 = acc0 + v * w0vecs[d // LANES][d % LANES]
            acc1 = acc1 + v * w1vecs[d // LANES][d % LANES]
        p0 = 1.0 / (1.0 + jnp.exp(-acc0))
        p1 = 1.0 / (1.0 + jnp.exp(-acc1))
        row_idx = g * LANES + iota
        plsc.store_scatter(out_v, [row_idx, jnp.zeros((LANES,), jnp.int32)], p0)
        plsc.store_scatter(out_v, [row_idx, jnp.ones((LANES,), jnp.int32)], p1)

    pltpu.sync_copy(out_v, out_hbm.at[pl.ds(base, BPW)])


@jax.jit
def _logreg_sc(idx3, table, wflat, bpad):
    mesh = plsc.VectorSubcoreMesh(core_axis_name="c", subcore_axis_name="s",
                                  num_cores=NC, num_subcores=NS)
    fn = pl.kernel(
        _sc_body,
        out_type=jax.ShapeDtypeStruct((B, 2), jnp.float32),
        mesh=mesh,
        compiler_params=pltpu.CompilerParams(needs_layout_passes=False,
                                             use_tc_tiling_on_sc=False),
        scratch_types=[
            pltpu.VMEM((NCHUNK, CHUNK), jnp.int32),
            pltpu.VMEM((TPAD, D), jnp.float32),
            pltpu.VMEM((BPW * 2 * D,), jnp.float32),
            pltpu.VMEM((2 * 2 * D,), jnp.float32),
            pltpu.VMEM((LANES,), jnp.float32),
            pltpu.VMEM((BPW, 2), jnp.float32),
            pltpu.SemaphoreType.DMA,
        ],
    )
    return fn(idx3, table, wflat, bpad)


def kernel(indices, embedding_matrix, dense_W, dense_b):
    idx = indices.reshape(B, T).astype(jnp.int32)
    idx3 = jnp.pad(idx, ((0, 0), (0, TPAD - T))).reshape(B, NCHUNK, CHUNK)
    wflat = dense_W.astype(jnp.float32).T.reshape(2 * 2 * D)
    bpad = jnp.pad(dense_b.astype(jnp.float32), (0, LANES - 2))
    return _logreg_sc(idx3, embedding_matrix, wflat, bpad)


# speedup vs baseline: 1.2634x; 1.0018x over previous
import functools

import jax
import jax.numpy as jnp
from jax import lax
from jax.experimental import pallas as pl
from jax.experimental.pallas import tpu as pltpu
from jax.experimental.pallas import tpu_sc as plsc

NC, NS, LANES = 2, 16, 16
NW = NC * NS
B = 1024
T = 1000
TPAD = 1024
CHUNK = 128
NCHUNK = TPAD // CHUNK
D = 32
BPW = B // NW
ROW_UNROLL = 8


def _sc_body(idx_hbm, table_hbm, w_hbm, b_hbm, out_hbm,
             idx_v, rows_a, rows_b, feat_v, w_v, bias_v, out_v, sem_a, sem_b):
    wid = lax.axis_index("s") * NC + lax.axis_index("c")
    base = wid * BPW
    iota = lax.iota(jnp.int32, LANES)

    pltpu.sync_copy(w_hbm, w_v)
    pltpu.sync_copy(b_hbm, bias_v)
    pltpu.sync_copy(idx_hbm.at[pl.ds(base, BPW)], idx_v)

    def fire(rows, sem_x, i):
        return [
            pltpu.async_copy(table_hbm.at[idx_v.at[i, j]],
                             rows.at[pl.ds(j * CHUNK, CHUNK)], sem_x)
            for j in range(NCHUNK)
        ]

    def drain(rows, sem_x, i):
        for j in range(NCHUNK):
            pltpu.make_async_copy(table_hbm.at[idx_v.at[i, j]],
                                  rows.at[pl.ds(j * CHUNK, CHUNK)], sem_x).wait()

    def compute(rows_v, i):
        zeros = jnp.zeros((LANES,), jnp.float32)
        init = (zeros, zeros, jnp.float32(-1.0), zeros, zeros)

        def row_body(it, c):
            s0, s1, m, bv0, bv1 = c
            for u in range(ROW_UNROLL):
                r = it * ROW_UNROLL + u
                a = rows_v[r, pl.ds(0, LANES)]
                bb = rows_v[r, pl.ds(LANES, LANES)]
                s0 = s0 + a
                s1 = s1 + bb
                nsq = jnp.sum(a * a + bb * bb)
                pred = nsq > m
                m = jnp.where(pred, nsq, m)
                pv = jnp.broadcast_to(pred, (LANES,))
                bv0 = jnp.where(pv, a, bv0)
                bv1 = jnp.where(pv, bb, bv1)
            return (s0, s1, m, bv0, bv1)

        s0, s1, m, bv0, bv1 = lax.fori_loop(0, T // ROW_UNROLL, row_body, init)
        inv = jnp.float32(1.0 / T)
        off = i * (2 * D)
        feat_v[pl.ds(off, LANES)] = s0 * inv
        feat_v[pl.ds(off + LANES, LANES)] = s1 * inv
        feat_v[pl.ds(off + 2 * LANES, LANES)] = bv0
        feat_v[pl.ds(off + 3 * LANES, LANES)] = bv1

    fire(rows_a, sem_a, 0)

    def pair_body(k, carry):
        i0 = 2 * k
        drain(rows_a, sem_a, i0)
        fire(rows_b, sem_b, i0 + 1)
        compute(rows_a, i0)
        drain(rows_b, sem_b, i0 + 1)

        @pl.when(k < BPW // 2 - 1)
        def _():
            fire(rows_a, sem_a, i0 + 2)

        compute(rows_b, i0 + 1)
        return carry

    lax.fori_loop(0, BPW // 2, pair_body, 0)

    bvec = bias_v[pl.ds(0, LANES)]
    b0 = bvec[0]
    b1 = bvec[1]
    w0vecs = [w_v[pl.ds(k * LANES, LANES)] for k in range(2 * D // LANES)]
    w1vecs = [w_v[pl.ds(2 * D + k * LANES, LANES)] for k in range(2 * D // LANES)]
    iota_feat = iota * (2 * D)
    for g in range(BPW // LANES):
        acc0 = jnp.broadcast_to(b0, (LANES,))
        acc1 = jnp.broadcast_to(b1, (LANES,))
        gbase = g * LANES * (2 * D)
        for d in range(2 * D):
            v = plsc.load_gather(feat_v, [iota_feat + (gbase + d)])
            acc0 = acc0 + v * w0vecs[d // LANES][d % LANES]
            acc1 = acc1 + v * w1vecs[d // LANES][d % LANES]
        p0 = 1.0 / (1.0 + jnp.exp(-acc0))
        p1 = 1.0 / (1.0 + jnp.exp(-acc1))
        row_idx = g * LANES + iota
        plsc.store_scatter(out_v, [row_idx, jnp.zeros((LANES,), jnp.int32)], p0)
        plsc.store_scatter(out_v, [row_idx, jnp.ones((LANES,), jnp.int32)], p1)

    pltpu.sync_copy(out_v, out_hbm.at[pl.ds(base, BPW)])


@jax.jit
def _logreg_sc(idx3, table, wflat, bpad):
    mesh = plsc.VectorSubcoreMesh(core_axis_name="c", subcore_axis_name="s",
                                  num_cores=NC, num_subcores=NS)
    fn = pl.kernel(
        _sc_body,
        out_type=jax.ShapeDtypeStruct((B, 2), jnp.float32),
        mesh=mesh,
        compiler_params=pltpu.CompilerParams(needs_layout_passes=False,
                                             use_tc_tiling_on_sc=False),
        scratch_types=[
            pltpu.VMEM((BPW, NCHUNK, CHUNK), jnp.int32),
            pltpu.VMEM((TPAD, D), jnp.float32),
            pltpu.VMEM((TPAD, D), jnp.float32),
            pltpu.VMEM((BPW * 2 * D,), jnp.float32),
            pltpu.VMEM((2 * 2 * D,), jnp.float32),
            pltpu.VMEM((LANES,), jnp.float32),
            pltpu.VMEM((BPW, 2), jnp.float32),
            pltpu.SemaphoreType.DMA,
            pltpu.SemaphoreType.DMA,
        ],
    )
    return fn(idx3, table, wflat, bpad)


def kernel(indices, embedding_matrix, dense_W, dense_b):
    idx = indices.reshape(B, T).astype(jnp.int32)
    idx3 = jnp.pad(idx, ((0, 0), (0, TPAD - T))).reshape(B, NCHUNK, CHUNK)
    wflat = dense_W.astype(jnp.float32).T.reshape(2 * 2 * D)
    bpad = jnp.pad(dense_b.astype(jnp.float32), (0, LANES - 2))
    return _logreg_sc(idx3, embedding_matrix, wflat, bpad)
